# baseline (speedup 1.0000x reference)
import functools

import jax
import jax.numpy as jnp
import numpy as np
from jax import lax
from jax.experimental import pallas as pl
from jax.experimental.pallas import tpu as pltpu
from jax.experimental.pallas import tpu_sc as plsc

B, T = 16384, 200
NBINS = 128
HIDDEN = 256
FEAT = 128

_info = plsc.get_sparse_core_info()
_NC, _NS = _info.num_cores, _info.num_subcores
_NW = _NC * _NS
_SB = 128
_ROWS_PER_W = B // _NW
_NSB = _ROWS_PER_W // _SB
_UNROLL = 32

def _make_harmony():
    return jax.random.uniform(jax.random.key(42), (B, 12), dtype=jnp.float32)


try:
    _HARMONY = np.asarray(_make_harmony())
except Exception:
    _HARMONY = None


def _sc_histogram(tokens_t, zeros2d, half, nrows):
    mesh = plsc.VectorSubcoreMesh(core_axis_name="c", subcore_axis_name="s")
    rows_per_w = nrows // _NW
    nsb = rows_per_w // _SB
    half_base = half * nrows

    @functools.partial(
        pl.kernel,
        mesh=mesh,
        out_type=[
            jax.ShapeDtypeStruct((nrows, NBINS), jnp.int32),
            jax.ShapeDtypeStruct((2, nrows), jnp.float32),
        ],
        scratch_types=[
            pltpu.VMEM((T, _SB), jnp.int32),
            pltpu.VMEM((T, _SB), jnp.int32),
            pltpu.VMEM((_SB, NBINS), jnp.int32),
            pltpu.VMEM((_SB, NBINS), jnp.int32),
            pltpu.VMEM((2, rows_per_w), jnp.float32),
            pltpu.SemaphoreType.DMA,
            pltpu.SemaphoreType.DMA,
            pltpu.SemaphoreType.DMA,
            pltpu.SemaphoreType.DMA,
            pltpu.SemaphoreType.DMA,
            pltpu.SemaphoreType.DMA,
        ],
        compiler_params=pltpu.CompilerParams(needs_layout_passes=False),
    )
    def hist_kernel(tok_hbm, zero_hbm, counts_hbm, stats_hbm,
                    tok_v0, tok_v1, hist_v0, hist_v1, stats_v,
                    tsem0, tsem1, zsem0, zsem1, osem0, osem1):
        wid = lax.axis_index("s") * _NC + lax.axis_index("c")
        lane = lax.iota(jnp.int32, 16)
        ones = jnp.ones((16,), jnp.int32)
        zero16 = jnp.zeros((16,), jnp.int32)
        tok_v = (tok_v0, tok_v1)
        hist_v = (hist_v0, hist_v1)
        tsem = (tsem0, tsem1)
        zsem = (zsem0, zsem1)
        osem = (osem0, osem1)
        wbase = wid * rows_per_w

        def start_in(sb):
            b = sb & 1
            ht = pltpu.async_copy(
                tok_hbm.at[:, pl.ds(half_base + wbase + sb * _SB, _SB)],
                tok_v[b], tsem[b])
            hz = pltpu.async_copy(zero_hbm, hist_v[b], zsem[b])
            return ht, hz

        pend_in = start_in(0)
        pend_out = [None, None]
        for sb in range(nsb):
            b = sb & 1
            ht, hz = pend_in
            ht.wait()
            hz.wait()
            if sb + 1 < nsb:
                if pend_out[1 - b] is not None:
                    pend_out[1 - b].wait()
                    pend_out[1 - b] = None
                pend_in = start_in(sb + 1)

            for blk in range(_SB // 16):
                rows = blk * 16 + lane

                @plsc.parallel_loop(0, T, unroll=_UNROLL, carry=(zero16, zero16))
                def t_body(t, carry):
                    s, ss = carry
                    v = tok_v[b][t, pl.ds(blk * 16, 16)]
                    p = jnp.bitwise_and(v, NBINS - 1)
                    plsc.addupdate_scatter(hist_v[b], [rows, p], ones)
                    return (s + v, ss + v * v)

                s, ss = t_body
                wrows = sb * _SB + rows
                plsc.store_scatter(stats_v, [0 * lane, wrows],
                                   s.astype(jnp.float32))
                plsc.store_scatter(stats_v, [0 * lane + 1, wrows],
                                   ss.astype(jnp.float32))

            pend_out[b] = pltpu.async_copy(
                hist_v[b], counts_hbm.at[pl.ds(wbase + sb * _SB, _SB)], osem[b])

        for b in (0, 1):
            if pend_out[b] is not None:
                pend_out[b].wait()
        pltpu.sync_copy(stats_v, stats_hbm.at[:, pl.ds(wbase, rows_per_w)])

    return hist_kernel(tokens_t, zeros2d)


_R = 4096


def _tc_mlp(counts, stats2, harmony, w1p, w1r, w1h, b1, w2, b2, nrows):
    def mlp_body(counts_ref, stats_ref, har_ref, w1p_ref, w1r_ref, w1h_ref,
                 b1_ref, w2_ref, b2_ref, out_ref):
        st = stats_ref[...]
        s = st[0:1, :]
        ss = st[1:2, :]
        mean = s * (1.0 / T)
        var = (ss - s * mean) * (1.0 / (T - 1))
        std = jnp.sqrt(jnp.maximum(var, 0.0))
        ms = jnp.concatenate([mean, std], axis=0)
        cn = counts_ref[...].astype(jnp.bfloat16)
        h = jnp.dot(cn, w1p_ref[...], preferred_element_type=jnp.float32)
        h += lax.dot_general(ms, w1r_ref[...], (((0,), (0,)), ((), ())),
                             preferred_element_type=jnp.float32)
        h += jnp.dot(har_ref[...], w1h_ref[...], preferred_element_type=jnp.float32)
        h += b1_ref[...]
        h = jnp.maximum(h, 0.0).astype(jnp.bfloat16)
        out_ref[...] = (
            jnp.dot(h, w2_ref[...], preferred_element_type=jnp.float32) + b2_ref[...]
        )

    return pl.pallas_call(
        mlp_body,
        grid=(nrows // _R,),
        in_specs=[
            pl.BlockSpec((_R, NBINS), lambda i: (i, 0)),
            pl.BlockSpec((2, _R), lambda i: (0, i)),
            pl.BlockSpec((_R, 12), lambda i: (i, 0)),
            pl.BlockSpec((NBINS, HIDDEN), lambda i: (0, 0)),
            pl.BlockSpec((2, HIDDEN), lambda i: (0, 0)),
            pl.BlockSpec((12, HIDDEN), lambda i: (0, 0)),
            pl.BlockSpec((1, HIDDEN), lambda i: (0, 0)),
            pl.BlockSpec((HIDDEN, FEAT), lambda i: (0, 0)),
            pl.BlockSpec((1, FEAT), lambda i: (0, 0)),
        ],
        out_specs=pl.BlockSpec((_R, FEAT), lambda i: (i, 0)),
        out_shape=jax.ShapeDtypeStruct((nrows, FEAT), jnp.float32),
    )(counts, stats2, harmony, w1p, w1r, w1h, b1, w2, b2)


def kernel(midi_tokens, W1, b1, W2, b2):
    zeros2d = jnp.zeros((_SB, NBINS), jnp.int32)
    tokens_t = midi_tokens.T
    harmony = jnp.asarray(_HARMONY) if _HARMONY is not None else _make_harmony()
    harmony = harmony.astype(jnp.bfloat16)
    w1p = (W1[:NBINS] * (1.0 / T)).astype(jnp.bfloat16)
    w1r = W1[NBINS:NBINS + 2]
    w1h = W1[NBINS + 10:NBINS + 22].astype(jnp.bfloat16)
    b1r = b1.reshape(1, HIDDEN)
    w2 = W2.astype(jnp.bfloat16)
    b2r = b2.reshape(1, FEAT)
    counts, stats2 = _sc_histogram(tokens_t, zeros2d, 0, B)
    return _tc_mlp(counts, stats2, harmony, w1p, w1r, w1h, b1r, w2, b2r, B)


# speedup vs baseline: 1.0542x; 1.0542x over previous
import functools

import jax
import jax.numpy as jnp
import numpy as np
from jax import lax
from jax.experimental import pallas as pl
from jax.experimental.pallas import tpu as pltpu
from jax.experimental.pallas import tpu_sc as plsc

B, T = 16384, 200
NBINS = 128
HIDDEN = 256
FEAT = 128

_info = plsc.get_sparse_core_info()
_NC, _NS = _info.num_cores, _info.num_subcores
_NW = _NC * _NS
_SB = 128
_ROWS_PER_W = B // _NW
_NSB = _ROWS_PER_W // _SB
_UNROLL = 16

def _make_harmony():
    return jax.random.uniform(jax.random.key(42), (B, 12), dtype=jnp.float32)


try:
    _HARMONY = np.asarray(_make_harmony())
except Exception:
    _HARMONY = None


def _sc_histogram(tokens_t, zeros2d, half, nrows):
    mesh = plsc.VectorSubcoreMesh(core_axis_name="c", subcore_axis_name="s")
    rows_per_w = nrows // _NW
    nsb = rows_per_w // _SB
    half_base = half * nrows

    @functools.partial(
        pl.kernel,
        mesh=mesh,
        out_type=[
            jax.ShapeDtypeStruct((nrows, NBINS), jnp.int32),
            jax.ShapeDtypeStruct((2, nrows), jnp.float32),
        ],
        scratch_types=[
            pltpu.VMEM((T, _SB), jnp.int32),
            pltpu.VMEM((T, _SB), jnp.int32),
            pltpu.VMEM((_SB, NBINS), jnp.int32),
            pltpu.VMEM((_SB, NBINS), jnp.int32),
            pltpu.VMEM((2, rows_per_w), jnp.float32),
            pltpu.SemaphoreType.DMA,
            pltpu.SemaphoreType.DMA,
            pltpu.SemaphoreType.DMA,
            pltpu.SemaphoreType.DMA,
            pltpu.SemaphoreType.DMA,
            pltpu.SemaphoreType.DMA,
        ],
        compiler_params=pltpu.CompilerParams(needs_layout_passes=False),
    )
    def hist_kernel(tok_hbm, zero_hbm, counts_hbm, stats_hbm,
                    tok_v0, tok_v1, hist_v0, hist_v1, stats_v,
                    tsem0, tsem1, zsem0, zsem1, osem0, osem1):
        wid = lax.axis_index("s") * _NC + lax.axis_index("c")
        lane = lax.iota(jnp.int32, 16)
        ones = jnp.ones((16,), jnp.int32)
        zero16 = jnp.zeros((16,), jnp.int32)
        tok_v = (tok_v0, tok_v1)
        hist_v = (hist_v0, hist_v1)
        tsem = (tsem0, tsem1)
        zsem = (zsem0, zsem1)
        osem = (osem0, osem1)
        wbase = wid * rows_per_w

        def start_in(sb):
            b = sb & 1
            ht = pltpu.async_copy(
                tok_hbm.at[:, pl.ds(half_base + wbase + sb * _SB, _SB)],
                tok_v[b], tsem[b])
            hz = pltpu.async_copy(zero_hbm, hist_v[b], zsem[b])
            return ht, hz

        pend_in = start_in(0)
        pend_out = [None, None]
        for sb in range(nsb):
            b = sb & 1
            ht, hz = pend_in
            ht.wait()
            hz.wait()
            if sb + 1 < nsb:
                if pend_out[1 - b] is not None:
                    pend_out[1 - b].wait()
                    pend_out[1 - b] = None
                pend_in = start_in(sb + 1)

            for blk in range(_SB // 16):
                rows = blk * 16 + lane

                @plsc.parallel_loop(0, T, unroll=_UNROLL, carry=(zero16, zero16))
                def t_body(t, carry):
                    s, ss = carry
                    v = tok_v[b][t, pl.ds(blk * 16, 16)]
                    p = jnp.bitwise_and(v, NBINS - 1)
                    plsc.addupdate_scatter(hist_v[b], [rows, p], ones)
                    return (s + v, ss + v * v)

                s, ss = t_body
                wrows = sb * _SB + rows
                plsc.store_scatter(stats_v, [0 * lane, wrows],
                                   s.astype(jnp.float32))
                plsc.store_scatter(stats_v, [0 * lane + 1, wrows],
                                   ss.astype(jnp.float32))

            pend_out[b] = pltpu.async_copy(
                hist_v[b], counts_hbm.at[pl.ds(wbase + sb * _SB, _SB)], osem[b])

        for b in (0, 1):
            if pend_out[b] is not None:
                pend_out[b].wait()
        pltpu.sync_copy(stats_v, stats_hbm.at[:, pl.ds(wbase, rows_per_w)])

    return hist_kernel(tokens_t, zeros2d)


_R = 4096


def _tc_mlp(counts, stats2, harmony, w1p, w1r, w1h, b1, w2, b2, nrows):
    def mlp_body(counts_ref, stats_ref, har_ref, w1p_ref, w1r_ref, w1h_ref,
                 b1_ref, w2_ref, b2_ref, out_ref):
        st = stats_ref[...]
        s = st[0:1, :]
        ss = st[1:2, :]
        mean = s * (1.0 / T)
        var = (ss - s * mean) * (1.0 / (T - 1))
        std = jnp.sqrt(jnp.maximum(var, 0.0))
        ms = jnp.concatenate([mean, std], axis=0)
        cn = counts_ref[...].astype(jnp.bfloat16)
        h = jnp.dot(cn, w1p_ref[...], preferred_element_type=jnp.float32)
        h += lax.dot_general(ms, w1r_ref[...], (((0,), (0,)), ((), ())),
                             preferred_element_type=jnp.float32)
        h += jnp.dot(har_ref[...], w1h_ref[...], preferred_element_type=jnp.float32)
        h += b1_ref[...]
        h = jnp.maximum(h, 0.0).astype(jnp.bfloat16)
        out_ref[...] = (
            jnp.dot(h, w2_ref[...], preferred_element_type=jnp.float32) + b2_ref[...]
        )

    return pl.pallas_call(
        mlp_body,
        grid=(nrows // _R,),
        in_specs=[
            pl.BlockSpec((_R, NBINS), lambda i: (i, 0)),
            pl.BlockSpec((2, _R), lambda i: (0, i)),
            pl.BlockSpec((_R, 12), lambda i: (i, 0)),
            pl.BlockSpec((NBINS, HIDDEN), lambda i: (0, 0)),
            pl.BlockSpec((2, HIDDEN), lambda i: (0, 0)),
            pl.BlockSpec((12, HIDDEN), lambda i: (0, 0)),
            pl.BlockSpec((1, HIDDEN), lambda i: (0, 0)),
            pl.BlockSpec((HIDDEN, FEAT), lambda i: (0, 0)),
            pl.BlockSpec((1, FEAT), lambda i: (0, 0)),
        ],
        out_specs=pl.BlockSpec((_R, FEAT), lambda i: (i, 0)),
        out_shape=jax.ShapeDtypeStruct((nrows, FEAT), jnp.float32),
    )(counts, stats2, harmony, w1p, w1r, w1h, b1, w2, b2)


def kernel(midi_tokens, W1, b1, W2, b2):
    zeros2d = jnp.zeros((_SB, NBINS), jnp.int32)
    tokens_t = midi_tokens.T
    harmony = jnp.asarray(_HARMONY) if _HARMONY is not None else _make_harmony()
    harmony = harmony.astype(jnp.bfloat16)
    w1p = (W1[:NBINS] * (1.0 / T)).astype(jnp.bfloat16)
    w1r = W1[NBINS:NBINS + 2]
    w1h = W1[NBINS + 10:NBINS + 22].astype(jnp.bfloat16)
    b1r = b1.reshape(1, HIDDEN)
    w2 = W2.astype(jnp.bfloat16)
    b2r = b2.reshape(1, FEAT)
    counts, stats2 = _sc_histogram(tokens_t, zeros2d, 0, B)
    return _tc_mlp(counts, stats2, harmony, w1p, w1r, w1h, b1r, w2, b2r, B)


# speedup vs baseline: 1.1891x; 1.1280x over previous
import functools

import jax
import jax.numpy as jnp
import numpy as np
from jax import lax
from jax.experimental import pallas as pl
from jax.experimental.pallas import tpu as pltpu
from jax.experimental.pallas import tpu_sc as plsc

B, T = 16384, 200
NBINS = 128
HIDDEN = 256
FEAT = 128

_info = plsc.get_sparse_core_info()
_NC, _NS = _info.num_cores, _info.num_subcores
_NW = _NC * _NS
_SB = 128
_ROWS_PER_W = B // _NW
_NSB = _ROWS_PER_W // _SB
_UNROLL = 16

def _make_harmony():
    return jax.random.uniform(jax.random.key(42), (B, 12), dtype=jnp.float32)


try:
    _HARMONY = np.asarray(_make_harmony())
except Exception:
    _HARMONY = None


def _sc_histogram(tokens_t, zeros2d, half, nrows):
    mesh = plsc.VectorSubcoreMesh(core_axis_name="c", subcore_axis_name="s")
    rows_per_w = nrows // _NW
    nsb = rows_per_w // _SB
    half_base = half * nrows

    @functools.partial(
        pl.kernel,
        mesh=mesh,
        out_type=[
            jax.ShapeDtypeStruct((nrows, NBINS), jnp.int32),
            jax.ShapeDtypeStruct((2, nrows), jnp.float32),
        ],
        scratch_types=[
            pltpu.VMEM((T, _SB), jnp.int32),
            pltpu.VMEM((T, _SB), jnp.int32),
            pltpu.VMEM((_SB, NBINS), jnp.int32),
            pltpu.VMEM((_SB, NBINS), jnp.int32),
            pltpu.VMEM((2, rows_per_w), jnp.float32),
            pltpu.SemaphoreType.DMA,
            pltpu.SemaphoreType.DMA,
            pltpu.SemaphoreType.DMA,
            pltpu.SemaphoreType.DMA,
            pltpu.SemaphoreType.DMA,
            pltpu.SemaphoreType.DMA,
        ],
        compiler_params=pltpu.CompilerParams(needs_layout_passes=False),
    )
    def hist_kernel(tok_hbm, zero_hbm, counts_hbm, stats_hbm,
                    tok_v0, tok_v1, hist_v0, hist_v1, stats_v,
                    tsem0, tsem1, zsem0, zsem1, osem0, osem1):
        wid = lax.axis_index("s") * _NC + lax.axis_index("c")
        lane = lax.iota(jnp.int32, 16)
        ones = jnp.ones((16,), jnp.int32)
        zero16 = jnp.zeros((16,), jnp.int32)
        tok_v = (tok_v0, tok_v1)
        hist_v = (hist_v0, hist_v1)
        tsem = (tsem0, tsem1)
        zsem = (zsem0, zsem1)
        osem = (osem0, osem1)
        wbase = wid * rows_per_w

        def start_in(sb):
            b = sb & 1
            ht = pltpu.async_copy(
                tok_hbm.at[:, pl.ds(half_base + wbase + sb * _SB, _SB)],
                tok_v[b], tsem[b])
            hz = pltpu.async_copy(
                zero_hbm.at[pl.ds(((wid + sb) % 4) * _SB, _SB)],
                hist_v[b], zsem[b])
            return ht, hz

        pend_in = start_in(0)
        pend_out = [None, None]
        for sb in range(nsb):
            b = sb & 1
            ht, hz = pend_in
            ht.wait()
            hz.wait()
            if sb + 1 < nsb:
                if pend_out[1 - b] is not None:
                    pend_out[1 - b].wait()
                    pend_out[1 - b] = None
                pend_in = start_in(sb + 1)

            for blk in range(_SB // 16):
                rows = blk * 16 + lane

                @plsc.parallel_loop(0, T, unroll=_UNROLL, carry=(zero16, zero16))
                def t_body(t, carry):
                    s, ss = carry
                    v = tok_v[b][t, pl.ds(blk * 16, 16)]
                    p = jnp.bitwise_and(v, NBINS - 1)
                    plsc.addupdate_scatter(hist_v[b], [rows, p], ones)
                    return (s + v, ss + v * v)

                s, ss = t_body
                wrows = sb * _SB + rows
                plsc.store_scatter(stats_v, [0 * lane, wrows],
                                   s.astype(jnp.float32))
                plsc.store_scatter(stats_v, [0 * lane + 1, wrows],
                                   ss.astype(jnp.float32))

            pend_out[b] = pltpu.async_copy(
                hist_v[b], counts_hbm.at[pl.ds(wbase + sb * _SB, _SB)], osem[b])

        for b in (0, 1):
            if pend_out[b] is not None:
                pend_out[b].wait()
        pltpu.sync_copy(stats_v, stats_hbm.at[:, pl.ds(wbase, rows_per_w)])

    return hist_kernel(tokens_t, zeros2d)


_R = 4096


def _tc_mlp(counts, stats2, harmony, w1p, w1r, w1h, b1, w2, b2, nrows):
    def mlp_body(counts_ref, stats_ref, har_ref, w1p_ref, w1r_ref, w1h_ref,
                 b1_ref, w2_ref, b2_ref, out_ref):
        st = stats_ref[...]
        s = st[0:1, :]
        ss = st[1:2, :]
        mean = s * (1.0 / T)
        var = (ss - s * mean) * (1.0 / (T - 1))
        std = jnp.sqrt(jnp.maximum(var, 0.0))
        ms = jnp.concatenate([mean, std], axis=0)
        cn = counts_ref[...].astype(jnp.bfloat16)
        h = jnp.dot(cn, w1p_ref[...], preferred_element_type=jnp.float32)
        h += lax.dot_general(ms, w1r_ref[...], (((0,), (0,)), ((), ())),
                             preferred_element_type=jnp.float32)
        h += jnp.dot(har_ref[...], w1h_ref[...], preferred_element_type=jnp.float32)
        h += b1_ref[...]
        h = jnp.maximum(h, 0.0).astype(jnp.bfloat16)
        out_ref[...] = (
            jnp.dot(h, w2_ref[...], preferred_element_type=jnp.float32) + b2_ref[...]
        )

    return pl.pallas_call(
        mlp_body,
        grid=(nrows // _R,),
        in_specs=[
            pl.BlockSpec((_R, NBINS), lambda i: (i, 0)),
            pl.BlockSpec((2, _R), lambda i: (0, i)),
            pl.BlockSpec((_R, 12), lambda i: (i, 0)),
            pl.BlockSpec((NBINS, HIDDEN), lambda i: (0, 0)),
            pl.BlockSpec((2, HIDDEN), lambda i: (0, 0)),
            pl.BlockSpec((12, HIDDEN), lambda i: (0, 0)),
            pl.BlockSpec((1, HIDDEN), lambda i: (0, 0)),
            pl.BlockSpec((HIDDEN, FEAT), lambda i: (0, 0)),
            pl.BlockSpec((1, FEAT), lambda i: (0, 0)),
        ],
        out_specs=pl.BlockSpec((_R, FEAT), lambda i: (i, 0)),
        out_shape=jax.ShapeDtypeStruct((nrows, FEAT), jnp.float32),
    )(counts, stats2, harmony, w1p, w1r, w1h, b1, w2, b2)


def kernel(midi_tokens, W1, b1, W2, b2):
    zeros2d = jnp.zeros((4 * _SB, NBINS), jnp.int32)
    tokens_t = midi_tokens.T
    harmony = jnp.asarray(_HARMONY) if _HARMONY is not None else _make_harmony()
    harmony = harmony.astype(jnp.bfloat16)
    w1p = (W1[:NBINS] * (1.0 / T)).astype(jnp.bfloat16)
    w1r = W1[NBINS:NBINS + 2]
    w1h = W1[NBINS + 10:NBINS + 22].astype(jnp.bfloat16)
    b1r = b1.reshape(1, HIDDEN)
    w2 = W2.astype(jnp.bfloat16)
    b2r = b2.reshape(1, FEAT)
    counts, stats2 = _sc_histogram(tokens_t, zeros2d, 0, B)
    return _tc_mlp(counts, stats2, harmony, w1p, w1r, w1h, b1r, w2, b2r, B)


# speedup vs baseline: 1.2142x; 1.0211x over previous
import functools

import jax
import jax.numpy as jnp
import numpy as np
from jax import lax
from jax.experimental import pallas as pl
from jax.experimental.pallas import tpu as pltpu
from jax.experimental.pallas import tpu_sc as plsc

B, T = 16384, 200
NBINS = 128
HIDDEN = 256
FEAT = 128

_info = plsc.get_sparse_core_info()
_NC, _NS = _info.num_cores, _info.num_subcores
_NW = _NC * _NS
_SB = 128
_ROWS_PER_W = B // _NW
_NSB = _ROWS_PER_W // _SB
_UNROLL = 16

def _make_harmony():
    return jax.random.uniform(jax.random.key(42), (B, 12), dtype=jnp.float32)


try:
    _HARMONY = np.asarray(_make_harmony())
except Exception:
    _HARMONY = None


def _sc_histogram(tokens_t, zeros2d, half, nrows):
    mesh = plsc.VectorSubcoreMesh(core_axis_name="c", subcore_axis_name="s")
    rows_per_w = nrows // _NW
    nsb = rows_per_w // _SB
    half_base = half * nrows

    @functools.partial(
        pl.kernel,
        mesh=mesh,
        out_type=[
            jax.ShapeDtypeStruct((nrows, NBINS), jnp.int32),
            jax.ShapeDtypeStruct((2, nrows), jnp.float32),
        ],
        scratch_types=[
            pltpu.VMEM((T, _SB), jnp.int32),
            pltpu.VMEM((T, _SB), jnp.int32),
            pltpu.VMEM((_SB, NBINS), jnp.int32),
            pltpu.VMEM((_SB, NBINS), jnp.int32),
            pltpu.VMEM((2, rows_per_w), jnp.float32),
            pltpu.SemaphoreType.DMA,
            pltpu.SemaphoreType.DMA,
            pltpu.SemaphoreType.DMA,
            pltpu.SemaphoreType.DMA,
            pltpu.SemaphoreType.DMA,
            pltpu.SemaphoreType.DMA,
        ],
        compiler_params=pltpu.CompilerParams(needs_layout_passes=False),
    )
    def hist_kernel(tok_hbm, zero_hbm, counts_hbm, stats_hbm,
                    tok_v0, tok_v1, hist_v0, hist_v1, stats_v,
                    tsem0, tsem1, zsem0, zsem1, osem0, osem1):
        wid = lax.axis_index("s") * _NC + lax.axis_index("c")
        lane = lax.iota(jnp.int32, 16)
        ones = jnp.ones((16,), jnp.int32)
        zero16 = jnp.zeros((16,), jnp.int32)
        tok_v = (tok_v0, tok_v1)
        hist_v = (hist_v0, hist_v1)
        tsem = (tsem0, tsem1)
        zsem = (zsem0, zsem1)
        osem = (osem0, osem1)
        wbase = wid * rows_per_w

        def start_in(sb):
            b = sb & 1
            ht = pltpu.async_copy(
                tok_hbm.at[:, pl.ds(half_base + wbase + sb * _SB, _SB)],
                tok_v[b], tsem[b])
            hz = pltpu.async_copy(
                zero_hbm.at[pl.ds(wid * _SB, _SB)],
                hist_v[b], zsem[b])
            return ht, hz

        pend_in = start_in(0)
        pend_out = [None, None]
        for sb in range(nsb):
            b = sb & 1
            ht, hz = pend_in
            ht.wait()
            hz.wait()
            if sb + 1 < nsb:
                if pend_out[1 - b] is not None:
                    pend_out[1 - b].wait()
                    pend_out[1 - b] = None
                pend_in = start_in(sb + 1)

            for blk in range(_SB // 16):
                rows = blk * 16 + lane

                @plsc.parallel_loop(0, T, unroll=_UNROLL, carry=(zero16, zero16))
                def t_body(t, carry):
                    s, ss = carry
                    v = tok_v[b][t, pl.ds(blk * 16, 16)]
                    p = jnp.bitwise_and(v, NBINS - 1)
                    plsc.addupdate_scatter(hist_v[b], [rows, p], ones)
                    return (s + v, ss + v * v)

                s, ss = t_body
                wrows = sb * _SB + rows
                plsc.store_scatter(stats_v, [0 * lane, wrows],
                                   s.astype(jnp.float32))
                plsc.store_scatter(stats_v, [0 * lane + 1, wrows],
                                   ss.astype(jnp.float32))

            pend_out[b] = pltpu.async_copy(
                hist_v[b], counts_hbm.at[pl.ds(wbase + sb * _SB, _SB)], osem[b])

        for b in (0, 1):
            if pend_out[b] is not None:
                pend_out[b].wait()
        pltpu.sync_copy(stats_v, stats_hbm.at[:, pl.ds(wbase, rows_per_w)])

    return hist_kernel(tokens_t, zeros2d)


_R = 4096


def _tc_mlp(counts, stats2, harmony, w1p, w1r, w1h, b1, w2, b2, nrows):
    def mlp_body(counts_ref, stats_ref, har_ref, w1p_ref, w1r_ref, w1h_ref,
                 b1_ref, w2_ref, b2_ref, out_ref):
        st = stats_ref[...]
        s = st[0:1, :]
        ss = st[1:2, :]
        mean = s * (1.0 / T)
        var = (ss - s * mean) * (1.0 / (T - 1))
        std = jnp.sqrt(jnp.maximum(var, 0.0))
        ms = jnp.concatenate([mean, std], axis=0)
        cn = counts_ref[...].astype(jnp.bfloat16)
        h = jnp.dot(cn, w1p_ref[...], preferred_element_type=jnp.float32)
        h += lax.dot_general(ms, w1r_ref[...], (((0,), (0,)), ((), ())),
                             preferred_element_type=jnp.float32)
        h += jnp.dot(har_ref[...], w1h_ref[...], preferred_element_type=jnp.float32)
        h += b1_ref[...]
        h = jnp.maximum(h, 0.0).astype(jnp.bfloat16)
        out_ref[...] = (
            jnp.dot(h, w2_ref[...], preferred_element_type=jnp.float32) + b2_ref[...]
        )

    return pl.pallas_call(
        mlp_body,
        grid=(nrows // _R,),
        in_specs=[
            pl.BlockSpec((_R, NBINS), lambda i: (i, 0)),
            pl.BlockSpec((2, _R), lambda i: (0, i)),
            pl.BlockSpec((_R, 12), lambda i: (i, 0)),
            pl.BlockSpec((NBINS, HIDDEN), lambda i: (0, 0)),
            pl.BlockSpec((2, HIDDEN), lambda i: (0, 0)),
            pl.BlockSpec((12, HIDDEN), lambda i: (0, 0)),
            pl.BlockSpec((1, HIDDEN), lambda i: (0, 0)),
            pl.BlockSpec((HIDDEN, FEAT), lambda i: (0, 0)),
            pl.BlockSpec((1, FEAT), lambda i: (0, 0)),
        ],
        out_specs=pl.BlockSpec((_R, FEAT), lambda i: (i, 0)),
        out_shape=jax.ShapeDtypeStruct((nrows, FEAT), jnp.float32),
    )(counts, stats2, harmony, w1p, w1r, w1h, b1, w2, b2)


def kernel(midi_tokens, W1, b1, W2, b2):
    zeros2d = jnp.zeros((_NW * _SB, NBINS), jnp.int32)
    tokens_t = midi_tokens.T
    harmony = jnp.asarray(_HARMONY) if _HARMONY is not None else _make_harmony()
    harmony = harmony.astype(jnp.bfloat16)
    w1p = (W1[:NBINS] * (1.0 / T)).astype(jnp.bfloat16)
    w1r = W1[NBINS:NBINS + 2]
    w1h = W1[NBINS + 10:NBINS + 22].astype(jnp.bfloat16)
    b1r = b1.reshape(1, HIDDEN)
    w2 = W2.astype(jnp.bfloat16)
    b2r = b2.reshape(1, FEAT)
    counts, stats2 = _sc_histogram(tokens_t, zeros2d, 0, B)
    return _tc_mlp(counts, stats2, harmony, w1p, w1r, w1h, b1r, w2, b2r, B)
